# baseline (speedup 1.0000x reference)
import functools

import jax
import jax.numpy as jnp
from jax import lax
from jax.experimental import pallas as pl
from jax.experimental.pallas import tpu as pltpu
from jax.experimental.pallas import tpu_sc as plsc

D = 64
NC = 2
NS = 16
NW = NC * NS
CHUNK = 128
NBUF = 4


def _emb_body(idx_hbm, table_hbm, out_hbm, idx_v, rows_v, gsem, wsem):
    nch = idx_hbm.shape[1]
    wid = lax.axis_index("s") * NC + lax.axis_index("c")

    pltpu.sync_copy(idx_hbm.at[wid], idx_v)

    def gather_start(c, b):
        pltpu.make_async_copy(
            table_hbm.at[idx_v.at[c]], rows_v.at[b], gsem.at[b]
        ).start()

    def gather_wait(c, b):
        pltpu.make_async_copy(
            table_hbm.at[idx_v.at[c]], rows_v.at[b], gsem.at[b]
        ).wait()

    def write_start(c, b):
        pltpu.make_async_copy(
            rows_v.at[b], out_hbm.at[wid, c], wsem.at[b]
        ).start()

    def write_wait(c, b):
        pltpu.make_async_copy(
            rows_v.at[b], out_hbm.at[wid, c], wsem.at[b]
        ).wait()

    for b in range(NBUF):
        gather_start(b, b)

    n_groups = nch // NBUF

    def group(g, carry):
        for b in range(NBUF):
            c = g * NBUF + b
            gather_wait(c, b)
            write_start(c, b)
            write_wait(c, b)
            nxt = c + NBUF

            @pl.when(nxt < nch)
            def _start_next():
                gather_start(nxt, b)

        return carry

    lax.fori_loop(0, n_groups, group, 0, unroll=False)


def kernel(x, table):
    B, S = x.shape
    total = B * S
    nch = total // (NW * CHUNK)
    idx = x.reshape(NW, nch, CHUNK).astype(jnp.int32)

    emb = pl.kernel(
        _emb_body,
        out_type=jax.ShapeDtypeStruct((NW, nch, CHUNK, D), jnp.float32),
        mesh=plsc.VectorSubcoreMesh(
            core_axis_name="c", subcore_axis_name="s",
            num_cores=NC, num_subcores=NS,
        ),
        scratch_types=[
            pltpu.VMEM((nch, CHUNK), jnp.int32),
            pltpu.VMEM((NBUF, CHUNK, D), jnp.float32),
            pltpu.SemaphoreType.DMA((NBUF,)),
            pltpu.SemaphoreType.DMA((NBUF,)),
        ],
        compiler_params=pltpu.CompilerParams(use_tc_tiling_on_sc=False),
    )
    out = emb(idx, table)
    return out.reshape(B, S, D)


# speedup vs baseline: 1.0011x; 1.0011x over previous
import functools

import jax
import jax.numpy as jnp
from jax import lax
from jax.experimental import pallas as pl
from jax.experimental.pallas import tpu as pltpu
from jax.experimental.pallas import tpu_sc as plsc

D = 64
NC = 2
NS = 16
NW = NC * NS
CHUNK = 256
NBUF = 5
LOOKA = 3


def _emb_body(idx_hbm, table_hbm, out_hbm, idx_v, rows_v, gsem, wsem):
    nch = idx_hbm.shape[1]
    wid = lax.axis_index("s") * NC + lax.axis_index("c")

    pltpu.sync_copy(idx_hbm.at[wid], idx_v)

    def gather_start(c, b):
        pltpu.make_async_copy(
            table_hbm.at[idx_v.at[c]], rows_v.at[b], gsem.at[b]
        ).start()

    def gather_wait(c, b):
        pltpu.make_async_copy(
            table_hbm.at[idx_v.at[c]], rows_v.at[b], gsem.at[b]
        ).wait()

    def write_start(c, b):
        pltpu.make_async_copy(
            rows_v.at[b], out_hbm.at[wid, c], wsem.at[b]
        ).start()

    def write_wait(c, b):
        pltpu.make_async_copy(
            rows_v.at[b], out_hbm.at[wid, c], wsem.at[b]
        ).wait()

    for c in range(LOOKA):
        gather_start(c, c % NBUF)

    n_groups = nch // NBUF

    def group(g, carry):
        for u in range(NBUF):
            c = g * NBUF + u
            b = u
            gather_wait(c, b)
            write_start(c, b)
            q = c + LOOKA
            bq = (u + LOOKA) % NBUF

            @pl.when(q < nch)
            def _arm_next():
                @pl.when(q >= NBUF)
                def _drain_old_write():
                    write_wait(q - NBUF, bq)

                gather_start(q, bq)

        return carry

    lax.fori_loop(0, n_groups, group, 0, unroll=False)

    for c in range(nch - NBUF, nch):
        write_wait(c, c % NBUF)


def kernel(x, table):
    B, S = x.shape
    total = B * S
    nch = total // (NW * CHUNK)
    idx = x.reshape(NW, nch, CHUNK).astype(jnp.int32)

    emb = pl.kernel(
        _emb_body,
        out_type=jax.ShapeDtypeStruct((NW, nch, CHUNK, D), jnp.float32),
        mesh=plsc.VectorSubcoreMesh(
            core_axis_name="c", subcore_axis_name="s",
            num_cores=NC, num_subcores=NS,
        ),
        scratch_types=[
            pltpu.VMEM((nch, CHUNK), jnp.int32),
            pltpu.VMEM((NBUF, CHUNK, D), jnp.float32),
            pltpu.SemaphoreType.DMA((NBUF,)),
            pltpu.SemaphoreType.DMA((NBUF,)),
        ],
        compiler_params=pltpu.CompilerParams(use_tc_tiling_on_sc=False),
    )
    out = emb(idx, table)
    return out.reshape(B, S, D)
